# baseline (speedup 1.0000x reference)
import functools

import jax
import jax.numpy as jnp
from jax import lax
from jax.experimental import pallas as pl
from jax.experimental.pallas import tpu as pltpu
from jax.experimental.pallas import tpu_sc as plsc

MODEL_DIM = 64
SCALE = 8.0

NC = 2
NS = 16
LANE = 16
IDX_W = 128
UNITS_PER_BATCH = 4
BATCH_ROWS = UNITS_PER_BATCH * IDX_W

S0 = 4096
S1 = 200
CBLK = S0 // IDX_W
W_S1 = 8
W_C = 4
S1_PER_W = S1 // W_S1
CG_PER_W = CBLK // W_C


def _body(idx_hbm, table_hbm, out_hbm, idx_v, rows_v, gsem):
    wid = lax.axis_index("s") * NC + lax.axis_index("c")
    ws1 = wid // W_C
    wc = wid % W_C
    s1_base = ws1 * S1_PER_W
    c_base = wc * CG_PER_W

    @pl.loop(0, S1_PER_W * (CG_PER_W // UNITS_PER_BATCH))
    def _batch(b):
        s1 = s1_base + b // 2
        c0 = c_base + (b % 2) * UNITS_PER_BATCH
        pltpu.sync_copy(idx_hbm.at[s1, pl.ds(c0 * IDX_W, BATCH_ROWS)], idx_v)
        descs = []
        for j in range(UNITS_PER_BATCH):
            descs.append(
                pltpu.async_copy(
                    table_hbm.at[idx_v.at[pl.ds(j * IDX_W, IDX_W)]],
                    rows_v.at[pl.ds(j * IDX_W, IDX_W)],
                    gsem,
                )
            )
        for d in descs:
            d.wait()

        @pl.loop(0, BATCH_ROWS)
        def _row(r):
            for k in range(MODEL_DIM // LANE):
                sl = pl.ds(k * LANE, LANE)
                rows_v[r, sl] = rows_v[r, sl] * SCALE

        for j in range(UNITS_PER_BATCH):
            pltpu.sync_copy(
                rows_v.at[pl.ds(j * IDX_W, IDX_W)],
                out_hbm.at[pl.ds((c0 + j) * IDX_W, IDX_W), s1],
            )


def kernel(x, table):
    xt = x.T.astype(jnp.int32)

    run = pl.kernel(
        _body,
        out_type=jax.ShapeDtypeStruct((S0, S1, MODEL_DIM), jnp.float32),
        mesh=plsc.VectorSubcoreMesh(core_axis_name="c", subcore_axis_name="s"),
        scratch_types=[
            pltpu.VMEM((BATCH_ROWS,), jnp.int32),
            pltpu.VMEM((BATCH_ROWS, MODEL_DIM), jnp.float32),
            pltpu.SemaphoreType.DMA,
        ],
        compiler_params=pltpu.CompilerParams(use_tc_tiling_on_sc=False),
    )
    return run(xt, table)
